# Initial kernel scaffold; baseline (speedup 1.0000x reference)
import jax
import jax.numpy as jnp
from jax.experimental import pallas as pl


def kernel(x_customer, x_fund, W_lin_c, b_lin_c, W_lin_f, b_lin_f, W_l_cf, b_l_cf, W_r_cf, W_l_fc, b_l_fc, W_r_fc, W_out, b_out, edge_attr_cf, edge_attr_fc, edge_src_cf, edge_dst_cf, edge_src_fc, edge_dst_fc):
    raise NotImplementedError("write your pallas kernel here")


# speedup vs baseline: 12.5135x; 12.5135x over previous
import jax
import jax.numpy as jnp
from jax import lax
from jax.experimental import pallas as pl
from jax.experimental.pallas import tpu as pltpu
from jax.experimental.pallas import tpu_sc as plsc

N_C = 100000
N_F = 10000
E = 600000
H = 128

NW = 32
CHUNK = 128
NCH = 150
IB = 15
NIB = NCH // IB
E_PAD = NW * NCH * CHUNK
N_FP = N_F + 8
N_CP = 100096
STRIPE = N_CP // 16
PW = 16


def _prep_body(xf, wf, bf, wl, wr, wo4, blfc, bo4, g_ref, b2_ref, c2_ref):
    f32 = jnp.float32
    a4 = jnp.dot(wl[...], wo4[...], preferred_element_type=f32)
    hf = jnp.maximum(xf[...] * wf[...] + bf[...], 0.0)
    ones_col = (lax.broadcasted_iota(jnp.int32, (N_F, PW), 1) == 2).astype(f32)
    g_ref[...] = jnp.dot(hf, a4, preferred_element_type=f32) + ones_col
    b2_ref[...] = jnp.dot(wr[...], wo4[...], preferred_element_type=f32)
    c2_ref[...] = jnp.dot(blfc[...], wo4[...], preferred_element_type=f32) + bo4[...]


def _prep(x_fund, W_lin_f, b_lin_f, W_l_fc, W_r_fc, W_out, b_l_fc, b_out):
    wo4 = jnp.pad(W_out, ((0, 0), (0, PW - 2)))
    bo4 = jnp.pad(b_out, (0, PW - 2)).reshape(1, PW)
    return pl.pallas_call(
        _prep_body,
        out_shape=[
            jax.ShapeDtypeStruct((N_F, PW), jnp.float32),
            jax.ShapeDtypeStruct((H, PW), jnp.float32),
            jax.ShapeDtypeStruct((1, PW), jnp.float32),
        ],
    )(x_fund, W_lin_f, b_lin_f.reshape(1, H), W_l_fc, W_r_fc, wo4,
      b_l_fc.reshape(1, H), bo4)


def _sc_body(g_hbm, src_hbm, dst_hbm, out_hbm,
             src_v, dst_v, rows_v, zbuf, acc, sem):
    c = lax.axis_index("c")
    s = lax.axis_index("s")
    wid = c * 16 + s

    zq = STRIPE // 16
    zv = jnp.zeros((PW,), jnp.float32)

    def zrow(i, carry):
        zbuf[i, :] = zv
        return carry

    lax.fori_loop(0, zq, zrow, 0)

    def zchunk(k, carry):
        pltpu.sync_copy(zbuf, acc.at[pl.ds(s * STRIPE + k * zq, zq)])
        return carry

    lax.fori_loop(0, 16, zchunk, 0)
    plsc.subcore_barrier()

    def outer(b, carry):
        pltpu.sync_copy(src_hbm.at[wid].at[pl.ds(b * IB, IB)], src_v)
        pltpu.sync_copy(dst_hbm.at[wid].at[pl.ds(b * IB, IB)], dst_v)

        def body(j, carry2):
            pltpu.async_copy(g_hbm.at[src_v.at[j]], rows_v, sem).wait()
            pltpu.sync_copy(rows_v, acc.at[dst_v.at[j]], add=True)
            return carry2

        lax.fori_loop(0, IB, body, 0)
        return carry

    lax.fori_loop(0, NIB, outer, 0)
    plsc.subcore_barrier()
    pltpu.sync_copy(acc.at[pl.ds(s * STRIPE, STRIPE)],
                    out_hbm.at[c].at[pl.ds(s * STRIPE, STRIPE)])


def _segment_accumulate(g4, src, dst):
    pad = E_PAD - E
    src3 = jnp.concatenate(
        [src, jnp.full((pad,), N_F, jnp.int32)]).reshape(NW, NCH, CHUNK)
    dst3 = jnp.concatenate(
        [dst, jnp.full((pad,), N_C, jnp.int32)]).reshape(NW, NCH, CHUNK)
    g4p = jnp.pad(g4, ((0, N_FP - N_F), (0, 0)))
    mesh = plsc.VectorSubcoreMesh(core_axis_name="c", subcore_axis_name="s",
                                  num_cores=2, num_subcores=16)
    run = pl.kernel(
        _sc_body,
        out_type=jax.ShapeDtypeStruct((2, N_CP, PW), jnp.float32),
        mesh=mesh,
        scratch_types=[
            pltpu.VMEM((IB, CHUNK), jnp.int32),
            pltpu.VMEM((IB, CHUNK), jnp.int32),
            pltpu.VMEM((CHUNK, PW), jnp.float32),
            pltpu.VMEM((STRIPE // 16, PW), jnp.float32),
            pltpu.VMEM_SHARED((N_CP, PW), jnp.float32),
            pltpu.SemaphoreType.DMA,
        ],
        compiler_params=pltpu.CompilerParams(use_tc_tiling_on_sc=False),
    )
    return run(g4p, src3, dst3)


def _main_body(x_ref, w_ref, b_ref, b2_ref, c2_ref, a_ref, o_ref):
    f32 = jnp.float32
    h = jnp.maximum(
        jnp.dot(x_ref[...], w_ref[...], preferred_element_type=f32) + b_ref[...],
        0.0)
    s = a_ref[0] + a_ref[1]
    cnt = jnp.maximum(s[:, 2:3], 1.0)
    res = s / cnt + jnp.dot(h, b2_ref[...], preferred_element_type=f32) + c2_ref[...]
    o_ref[...] = res[:, 0:2]


def _main(x_customer, W_lin_c, b_lin_c, b24, c24, acc):
    R = 2000
    grid = (N_C // R,)
    return pl.pallas_call(
        _main_body,
        grid=grid,
        in_specs=[
            pl.BlockSpec((R, 101), lambda i: (i, 0)),
            pl.BlockSpec((101, H), lambda i: (0, 0)),
            pl.BlockSpec((1, H), lambda i: (0, 0)),
            pl.BlockSpec((H, PW), lambda i: (0, 0)),
            pl.BlockSpec((1, PW), lambda i: (0, 0)),
            pl.BlockSpec((2, R, PW), lambda i: (0, i, 0)),
        ],
        out_specs=pl.BlockSpec((R, 2), lambda i: (i, 0)),
        out_shape=jax.ShapeDtypeStruct((N_C, 2), jnp.float32),
    )(x_customer, W_lin_c, b_lin_c.reshape(1, H), b24, c24, acc)


def kernel(x_customer, x_fund, W_lin_c, b_lin_c, W_lin_f, b_lin_f,
           W_l_cf, b_l_cf, W_r_cf, W_l_fc, b_l_fc, W_r_fc, W_out, b_out,
           edge_attr_cf, edge_attr_fc, edge_src_cf, edge_dst_cf,
           edge_src_fc, edge_dst_fc):
    g4, b24, c24 = _prep(x_fund, W_lin_f, b_lin_f, W_l_fc, W_r_fc, W_out,
                         b_l_fc, b_out)
    acc = _segment_accumulate(g4, edge_src_fc, edge_dst_fc)
    return _main(x_customer, W_lin_c, b_lin_c, b24, c24, acc)
